# baseline (speedup 1.0000x reference)
import functools

import jax
import jax.numpy as jnp
from jax import lax
from jax.experimental import pallas as pl
from jax.experimental.pallas import tpu as pltpu
from jax.experimental.pallas import tpu_sc as plsc

N_SEQ_LEN = 20
NUM_EMBEDDINGS = 100000
EMBEDDING_DIM = 32
BATCH = 4096

_LANES = 16
_NW = 32
_ROWS_TOTAL = BATCH * N_SEQ_LEN
_ROWS_PER_W = _ROWS_TOTAL // _NW
_CHUNK = 128
_CHUNKS_PER_W = _ROWS_PER_W // _CHUNK


def _body(idx_hbm, tab_hbm, out_hbm, idx_v, rows_v, sem):
    nc = 2
    wid = lax.axis_index("s") * nc + lax.axis_index("c")
    base = wid * _CHUNKS_PER_W

    pltpu.sync_copy(idx_hbm.at[wid], idx_v)

    lane = lax.iota(jnp.int32, _LANES)
    for c in range(_CHUNKS_PER_W):
        for s in range(_CHUNK // _LANES):
            start = (c * _CHUNK + s * _LANES) % N_SEQ_LEN
            t = lane + start
            d = jnp.where(t >= N_SEQ_LEN, t - N_SEQ_LEN, t)
            sl = (c, pl.ds(s * _LANES, _LANES))
            idx_v[sl] = idx_v[sl] + d * NUM_EMBEDDINGS

    copies = [
        pltpu.async_copy(tab_hbm.at[idx_v.at[c]], rows_v.at[c], sem)
        for c in range(_CHUNKS_PER_W)
    ]
    for cp in copies:
        cp.wait()

    pltpu.sync_copy(rows_v, out_hbm.at[pl.ds(base, _CHUNKS_PER_W)])


@jax.jit
def kernel(x, tables):
    idx = x.astype(jnp.int32).reshape(_NW, _CHUNKS_PER_W, _CHUNK)
    tab = tables.reshape(N_SEQ_LEN * NUM_EMBEDDINGS, EMBEDDING_DIM)

    mesh = plsc.VectorSubcoreMesh(core_axis_name="c", subcore_axis_name="s")
    run = pl.kernel(
        _body,
        mesh=mesh,
        compiler_params=pltpu.CompilerParams(use_tc_tiling_on_sc=False),
        out_type=jax.ShapeDtypeStruct(
            (_NW * _CHUNKS_PER_W, _CHUNK, EMBEDDING_DIM), jnp.float32
        ),
        scratch_types=[
            pltpu.VMEM((_CHUNKS_PER_W, _CHUNK), jnp.int32),
            pltpu.VMEM((_CHUNKS_PER_W, _CHUNK, EMBEDDING_DIM), jnp.float32),
            pltpu.SemaphoreType.DMA,
        ],
    )
    out = run(idx, tab)
    return out.reshape(BATCH, N_SEQ_LEN, EMBEDDING_DIM)
